# Initial kernel scaffold; baseline (speedup 1.0000x reference)
import jax
import jax.numpy as jnp
from jax.experimental import pallas as pl


def kernel(x, edge_index, W1_l, b1, W1_r, W2_l, b2, W2_r):
    raise NotImplementedError("write your pallas kernel here")


# speedup vs baseline: 11.8658x; 11.8658x over previous
import functools

import jax
import jax.numpy as jnp
from jax import lax
from jax.experimental import pallas as pl
from jax.experimental.pallas import tpu as pltpu
from jax.experimental.pallas import tpu_sc as plsc

NC = 2
NS = 16
NW = NC * NS

CH = 80
RW = 16

_SC_PARAMS = pltpu.CompilerParams(use_tc_tiling_on_sc=False)


def _sc_edge_pass(table, g2d, s2d, zeros, n, nchunk):
    rpw = nchunk // NW
    rpl = 25
    outer = rpw // rpl
    mesh = plsc.VectorSubcoreMesh(core_axis_name="c", subcore_axis_name="s")

    @functools.partial(
        pl.kernel,
        out_type=jax.ShapeDtypeStruct((NC, n, RW), jnp.float32),
        mesh=mesh,
        scratch_types=[
            pltpu.VMEM((rpl, CH), jnp.int32),
            pltpu.VMEM((rpl, CH), jnp.int32),
            pltpu.VMEM((CH, RW), jnp.float32),
            pltpu.VMEM_SHARED((n, RW), jnp.float32),
            pltpu.SemaphoreType.DMA,
        ],
        compiler_params=_SC_PARAMS,
    )
    def k(t_hbm, g_hbm, s_hbm, z_hbm, out_hbm, gb, sb, rows, acc, sem):
        cid = lax.axis_index("c")
        sid = lax.axis_index("s")

        @pl.when(sid == 0)
        def _():
            pltpu.sync_copy(z_hbm, acc)

        plsc.subcore_barrier()
        wid = cid * NS + sid

        @pl.loop(0, outer)
        def _(o):
            base = wid * rpw + o * rpl
            pltpu.sync_copy(g_hbm.at[pl.ds(base, rpl)], gb)
            pltpu.sync_copy(s_hbm.at[pl.ds(base, rpl)], sb)

            @pl.loop(0, rpl)
            def _(j):
                pltpu.async_copy(t_hbm.at[gb.at[j]], rows, sem).wait()
                pltpu.sync_copy(rows, acc.at[sb.at[j]], add=True)

        plsc.subcore_barrier()

        @pl.when(sid == 0)
        def _():
            pltpu.sync_copy(acc, out_hbm.at[cid])

    return k(table, g2d, s2d, zeros)


def _tc_combine(parts, n, tile):
    nt = n // tile

    def body(p_ref, r_ref):
        p = p_ref[...]
        cnt = p[0, :, 3] + p[1, :, 3]
        r = 1.0 / jnp.maximum(cnt, 1.0)
        col0 = lax.broadcasted_iota(jnp.int32, (tile, RW), 1) == 0
        r_ref[...] = jnp.where(col0, r[:, None], 0.0)

    return pl.pallas_call(
        body,
        grid=(nt,),
        in_specs=[pl.BlockSpec((NC, tile, RW), lambda i: (0, i, 0))],
        out_specs=pl.BlockSpec((tile, RW), lambda i: (i, 0)),
        out_shape=jax.ShapeDtypeStruct((n, RW), jnp.float32),
    )(parts)


def _tc_final(parts, x, a2, w1l, w1r, b1, w2l, w2r, b2, n, tile):
    nt = n // tile
    inv_n = 1.0 / n

    def body(p_ref, x_ref, a_ref, w1l_ref, w1r_ref, b1_ref, w2l_ref, w2r_ref,
             b2_ref, o_ref, acc_ref):
        i = pl.program_id(0)

        @pl.when(i == 0)
        def _():
            acc_ref[...] = jnp.zeros_like(acc_ref)

        p = p_ref[...]
        a = a_ref[0, :, 0] + a_ref[1, :, 0]
        cnt = p[0, :, 3] + p[1, :, 3]
        s1 = p[0, :, :3] + p[1, :, :3]
        mean1 = s1 / jnp.maximum(cnt, 1.0)[:, None]
        pre = (
            jnp.dot(mean1, w1l_ref[...], preferred_element_type=jnp.float32,
                    precision=lax.Precision.HIGHEST)
            + jnp.dot(x_ref[...], w1r_ref[...],
                      preferred_element_type=jnp.float32,
                      precision=lax.Precision.HIGHEST)
            + b1_ref[...]
        )
        h = jnp.maximum(pre, 0.0)
        aw = jnp.concatenate([a[None, :], jnp.ones_like(a)[None, :]], axis=0)
        acc_ref[0:2, :] += jnp.dot(aw, h, preferred_element_type=jnp.float32,
                                   precision=lax.Precision.HIGHEST)

        @pl.when(i == nt - 1)
        def _():
            sa = acc_ref[0:1, :] * inv_n
            sh = acc_ref[1:2, :] * inv_n
            o_ref[...] = (
                jnp.dot(sa, w2l_ref[...], preferred_element_type=jnp.float32,
                        precision=lax.Precision.HIGHEST)
                + jnp.dot(sh, w2r_ref[...], preferred_element_type=jnp.float32,
                          precision=lax.Precision.HIGHEST)
                + b2_ref[...]
            )

    hid = w1l.shape[1]
    out = w2l.shape[1]
    return pl.pallas_call(
        body,
        grid=(nt,),
        in_specs=[
            pl.BlockSpec((NC, tile, RW), lambda i: (0, i, 0)),
            pl.BlockSpec((tile, 3), lambda i: (i, 0)),
            pl.BlockSpec((NC, tile, RW), lambda i: (0, i, 0)),
            pl.BlockSpec((3, hid), lambda i: (0, 0)),
            pl.BlockSpec((3, hid), lambda i: (0, 0)),
            pl.BlockSpec((1, hid), lambda i: (0, 0)),
            pl.BlockSpec((hid, out), lambda i: (0, 0)),
            pl.BlockSpec((hid, out), lambda i: (0, 0)),
            pl.BlockSpec((1, out), lambda i: (0, 0)),
        ],
        out_specs=pl.BlockSpec((1, out), lambda i: (0, 0)),
        out_shape=jax.ShapeDtypeStruct((1, out), jnp.float32),
        scratch_shapes=[pltpu.VMEM((8, hid), jnp.float32)],
    )(parts, x, a2, w1l, w1r, b1, w2l, w2r, b2)


@jax.jit
def kernel(x, edge_index, W1_l, b1, W1_r, W2_l, b2, W2_r):
    n = x.shape[0]
    e = edge_index.shape[1]
    nchunk = e // CH
    tile = 2000

    src2d = edge_index[0].reshape(nchunk, CH)
    dst2d = edge_index[1].reshape(nchunk, CH)
    x16 = jnp.concatenate(
        [x, jnp.ones((n, 1), x.dtype), jnp.zeros((n, RW - 4), x.dtype)], axis=1)
    zeros16 = jnp.zeros((n, RW), jnp.float32)

    parts = _sc_edge_pass(x16, src2d, dst2d, zeros16, n, nchunk)
    r16 = _tc_combine(parts, n, tile)
    a2 = _sc_edge_pass(r16, dst2d, src2d, zeros16, n, nchunk)
    out = _tc_final(parts, x, a2, W1_l.T, W1_r.T, b1[None, :],
                    W2_l.T, W2_r.T, b2[None, :], n, tile)
    return out.reshape(-1)


# speedup vs baseline: 23.8905x; 2.0134x over previous
import functools

import jax
import jax.numpy as jnp
from jax import lax
from jax.experimental import pallas as pl
from jax.experimental.pallas import tpu as pltpu
from jax.experimental.pallas import tpu_sc as plsc

NC = 2
NS = 16
NW = NC * NS

CH = 80
RW = 16

_SC_PARAMS = pltpu.CompilerParams(use_tc_tiling_on_sc=False)


def _sc_edge_pass(table, g2d, s2d, zeros, n, nchunk):
    rpw = nchunk // NW
    rpl = 125
    outer = rpw // rpl
    nbuf = 5
    grps = rpl // nbuf
    mesh = plsc.VectorSubcoreMesh(core_axis_name="c", subcore_axis_name="s")

    @functools.partial(
        pl.kernel,
        out_type=jax.ShapeDtypeStruct((NC, n, RW), jnp.float32),
        mesh=mesh,
        scratch_types=[
            pltpu.VMEM((rpl, CH), jnp.int32),
            pltpu.VMEM((rpl, CH), jnp.int32),
            [pltpu.VMEM((CH, RW), jnp.float32)] * nbuf,
            pltpu.VMEM_SHARED((n, RW), jnp.float32),
            [pltpu.SemaphoreType.DMA] * nbuf,
            [pltpu.SemaphoreType.DMA] * nbuf,
        ],
        compiler_params=_SC_PARAMS,
    )
    def k(t_hbm, g_hbm, s_hbm, z_hbm, out_hbm, gb, sb, rows, acc, gsem, ssem):
        cid = lax.axis_index("c")
        sid = lax.axis_index("s")

        @pl.when(sid == 0)
        def _():
            pltpu.sync_copy(z_hbm, acc)

        plsc.subcore_barrier()
        wid = cid * NS + sid

        @pl.loop(0, outer)
        def _(o):
            base = wid * rpw + o * rpl
            pltpu.sync_copy(g_hbm.at[pl.ds(base, rpl)], gb)
            pltpu.sync_copy(s_hbm.at[pl.ds(base, rpl)], sb)

            @pl.loop(0, grps)
            def _(g):
                not_first = (o > 0) | (g > 0)
                descs = []
                for b in range(nbuf):
                    @pl.when(not_first)
                    def _(b=b):
                        pltpu.make_async_copy(
                            t_hbm.at[pl.ds(0, CH)], rows[b], ssem[b]).wait()
                    descs.append(pltpu.async_copy(
                        t_hbm.at[gb.at[g * nbuf + b]], rows[b], gsem[b]))
                for b in range(nbuf):
                    descs[b].wait()
                    pltpu.async_copy(rows[b], acc.at[sb.at[g * nbuf + b]],
                                     ssem[b], add=True)

        for b in range(nbuf):
            pltpu.make_async_copy(t_hbm.at[pl.ds(0, CH)], rows[b],
                                  ssem[b]).wait()

        plsc.subcore_barrier()

        @pl.when(sid == 0)
        def _():
            pltpu.sync_copy(acc, out_hbm.at[cid])

    return k(table, g2d, s2d, zeros)


def _tc_combine(parts, n, tile):
    nt = n // tile

    def body(p_ref, r_ref):
        p = p_ref[...]
        cnt = p[0, :, 3] + p[1, :, 3]
        r = 1.0 / jnp.maximum(cnt, 1.0)
        col0 = lax.broadcasted_iota(jnp.int32, (tile, RW), 1) == 0
        r_ref[...] = jnp.where(col0, r[:, None], 0.0)

    return pl.pallas_call(
        body,
        grid=(nt,),
        in_specs=[pl.BlockSpec((NC, tile, RW), lambda i: (0, i, 0))],
        out_specs=pl.BlockSpec((tile, RW), lambda i: (i, 0)),
        out_shape=jax.ShapeDtypeStruct((n, RW), jnp.float32),
    )(parts)


def _tc_final(parts, x, a2, w1l, w1r, b1, w2l, w2r, b2, n, tile):
    nt = n // tile
    inv_n = 1.0 / n

    def body(p_ref, x_ref, a_ref, w1l_ref, w1r_ref, b1_ref, w2l_ref, w2r_ref,
             b2_ref, o_ref, acc_ref):
        i = pl.program_id(0)

        @pl.when(i == 0)
        def _():
            acc_ref[...] = jnp.zeros_like(acc_ref)

        p = p_ref[...]
        a = a_ref[0, :, 0] + a_ref[1, :, 0]
        cnt = p[0, :, 3] + p[1, :, 3]
        s1 = p[0, :, :3] + p[1, :, :3]
        mean1 = s1 / jnp.maximum(cnt, 1.0)[:, None]
        pre = (
            jnp.dot(mean1, w1l_ref[...], preferred_element_type=jnp.float32,
                    precision=lax.Precision.HIGHEST)
            + jnp.dot(x_ref[...], w1r_ref[...],
                      preferred_element_type=jnp.float32,
                      precision=lax.Precision.HIGHEST)
            + b1_ref[...]
        )
        h = jnp.maximum(pre, 0.0)
        aw = jnp.concatenate([a[None, :], jnp.ones_like(a)[None, :]], axis=0)
        acc_ref[0:2, :] += jnp.dot(aw, h, preferred_element_type=jnp.float32,
                                   precision=lax.Precision.HIGHEST)

        @pl.when(i == nt - 1)
        def _():
            sa = acc_ref[0:1, :] * inv_n
            sh = acc_ref[1:2, :] * inv_n
            o_ref[...] = (
                jnp.dot(sa, w2l_ref[...], preferred_element_type=jnp.float32,
                        precision=lax.Precision.HIGHEST)
                + jnp.dot(sh, w2r_ref[...], preferred_element_type=jnp.float32,
                          precision=lax.Precision.HIGHEST)
                + b2_ref[...]
            )

    hid = w1l.shape[1]
    out = w2l.shape[1]
    return pl.pallas_call(
        body,
        grid=(nt,),
        in_specs=[
            pl.BlockSpec((NC, tile, RW), lambda i: (0, i, 0)),
            pl.BlockSpec((tile, 3), lambda i: (i, 0)),
            pl.BlockSpec((NC, tile, RW), lambda i: (0, i, 0)),
            pl.BlockSpec((3, hid), lambda i: (0, 0)),
            pl.BlockSpec((3, hid), lambda i: (0, 0)),
            pl.BlockSpec((1, hid), lambda i: (0, 0)),
            pl.BlockSpec((hid, out), lambda i: (0, 0)),
            pl.BlockSpec((hid, out), lambda i: (0, 0)),
            pl.BlockSpec((1, out), lambda i: (0, 0)),
        ],
        out_specs=pl.BlockSpec((1, out), lambda i: (0, 0)),
        out_shape=jax.ShapeDtypeStruct((1, out), jnp.float32),
        scratch_shapes=[pltpu.VMEM((8, hid), jnp.float32)],
    )(parts, x, a2, w1l, w1r, b1, w2l, w2r, b2)


@jax.jit
def kernel(x, edge_index, W1_l, b1, W1_r, W2_l, b2, W2_r):
    n = x.shape[0]
    e = edge_index.shape[1]
    nchunk = e // CH
    tile = 2000

    src2d = edge_index[0].reshape(nchunk, CH)
    dst2d = edge_index[1].reshape(nchunk, CH)
    x16 = jnp.concatenate(
        [x, jnp.ones((n, 1), x.dtype), jnp.zeros((n, RW - 4), x.dtype)], axis=1)
    zeros16 = jnp.zeros((n, RW), jnp.float32)

    parts = _sc_edge_pass(x16, src2d, dst2d, zeros16, n, nchunk)
    r16 = _tc_combine(parts, n, tile)
    a2 = _sc_edge_pass(r16, dst2d, src2d, zeros16, n, nchunk)
    out = _tc_final(parts, x, a2, W1_l.T, W1_r.T, b1[None, :],
                    W2_l.T, W2_r.T, b2[None, :], n, tile)
    return out.reshape(-1)
